# baseline (speedup 1.0000x reference)
import functools

import jax
import jax.numpy as jnp
from jax import lax
from jax.experimental import pallas as pl
from jax.experimental.pallas import tpu as pltpu
from jax.experimental.pallas import tpu_sc as plsc

_D = 2048
_NTOK = 32768
_NC = 2
_NS = 16
_NW = _NC * _NS
_BPW = _NTOK // _NW
_C = 16
_NBUF = 3
_NCHUNK = _BPW // _C
_NGRP = _NCHUNK // _NBUF
_NTAIL = _NCHUNK - _NGRP * _NBUF


def _make_gather():
    mesh = plsc.VectorSubcoreMesh(core_axis_name="c", subcore_axis_name="s")

    @functools.partial(
        pl.kernel,
        mesh=mesh,
        out_type=jax.ShapeDtypeStruct((_NTOK, _D), jnp.float32),
        scratch_types=(
            [pltpu.VMEM((_NCHUNK, _C), jnp.int32)]
            + [pltpu.VMEM((_C, _D), jnp.float32) for _ in range(_NBUF)]
            + [pltpu.SemaphoreType.DMA for _ in range(2 * _NBUF)]
        ),
    )
    def gather_kernel(table_hbm, idx_hbm, out_hbm, idx_v, *bufs_and_sems):
        rows = list(bufs_and_sems[:_NBUF])
        gsem = list(bufs_and_sems[_NBUF:2 * _NBUF])
        ssem = list(bufs_and_sems[2 * _NBUF:])
        cid = lax.axis_index("c")
        sid = lax.axis_index("s")
        wid = sid * _NC + cid
        pltpu.sync_copy(idx_hbm.at[pl.ds(wid * _NCHUNK, _NCHUNK)], idx_v)
        base = wid * _BPW

        def gather_chunk(i, b):
            return pltpu.make_async_copy(
                table_hbm.at[idx_v.at[i]], rows[b], gsem[b]
            )

        def store_chunk(i, b):
            return pltpu.make_async_copy(
                rows[b], out_hbm.at[pl.ds(base + i * _C, _C)], ssem[b]
            )

        for b in range(_NBUF):
            gather_chunk(b, b).start()

        def group(g, carry):
            i0 = g * _NBUF
            for b in range(_NBUF):
                gather_chunk(i0 + b, b).wait()
                store_chunk(i0 + b, b).start()
            for b in range(_NBUF):
                store_chunk(i0 + b, b).wait()
                gather_chunk(i0 + _NBUF + b, b).start()
            return carry

        lax.fori_loop(0, _NGRP - 1, group, 0)

        i0 = (_NGRP - 1) * _NBUF
        for b in range(_NBUF):
            gather_chunk(i0 + b, b).wait()
            store_chunk(i0 + b, b).start()
        for b in range(_NBUF):
            store_chunk(i0 + b, b).wait()
        for t in range(_NTAIL):
            i = _NGRP * _NBUF + t
            gather_chunk(i, t).start()
        for t in range(_NTAIL):
            i = _NGRP * _NBUF + t
            gather_chunk(i, t).wait()
            store_chunk(i, t).start()
        for t in range(_NTAIL):
            i = _NGRP * _NBUF + t
            store_chunk(i, t).wait()

    return gather_kernel


_gather = _make_gather()


@jax.jit
def _lookup(table, idx2):
    return _gather(table, idx2)


def kernel(input_ids, embed_weight):
    idx_flat = input_ids.reshape(-1).astype(jnp.int32)
    idx2 = idx_flat.reshape(_NW * _NCHUNK, _C)
    out = _lookup(embed_weight, idx2)
    return out.reshape(input_ids.shape + (embed_weight.shape[-1],))


# speedup vs baseline: 1.0119x; 1.0119x over previous
import functools

import jax
import jax.numpy as jnp
from jax import lax
from jax.experimental import pallas as pl
from jax.experimental.pallas import tpu as pltpu
from jax.experimental.pallas import tpu_sc as plsc

_D = 2048
_NTOK = 32768
_NC = 2
_NS = 16
_NW = _NC * _NS
_BPW = _NTOK // _NW
_C = 8
_NBUF = 6
_NCHUNK = _BPW // _C
_NGRP = _NCHUNK // _NBUF
_NTAIL = _NCHUNK - _NGRP * _NBUF


def _make_gather():
    mesh = plsc.VectorSubcoreMesh(core_axis_name="c", subcore_axis_name="s")

    @functools.partial(
        pl.kernel,
        mesh=mesh,
        out_type=jax.ShapeDtypeStruct((_NTOK, _D), jnp.float32),
        scratch_types=(
            [pltpu.VMEM((_NCHUNK, _C), jnp.int32)]
            + [pltpu.VMEM((_C, _D), jnp.float32) for _ in range(_NBUF)]
            + [pltpu.SemaphoreType.DMA for _ in range(2 * _NBUF)]
        ),
    )
    def gather_kernel(table_hbm, idx_hbm, out_hbm, idx_v, *bufs_and_sems):
        rows = list(bufs_and_sems[:_NBUF])
        gsem = list(bufs_and_sems[_NBUF:2 * _NBUF])
        ssem = list(bufs_and_sems[2 * _NBUF:])
        cid = lax.axis_index("c")
        sid = lax.axis_index("s")
        wid = sid * _NC + cid
        pltpu.sync_copy(idx_hbm.at[pl.ds(wid * _NCHUNK, _NCHUNK)], idx_v)
        base = wid * _BPW

        def gather_chunk(i, b):
            return pltpu.make_async_copy(
                table_hbm.at[idx_v.at[i]], rows[b], gsem[b]
            )

        def store_chunk(i, b):
            return pltpu.make_async_copy(
                rows[b], out_hbm.at[pl.ds(base + i * _C, _C)], ssem[b]
            )

        for b in range(_NBUF):
            gather_chunk(b, b).start()

        def group(g, carry):
            i0 = g * _NBUF
            for b in range(_NBUF):
                gather_chunk(i0 + b, b).wait()
                store_chunk(i0 + b, b).start()
            for b in range(_NBUF):
                store_chunk(i0 + b, b).wait()
                gather_chunk(i0 + _NBUF + b, b).start()
            return carry

        lax.fori_loop(0, _NGRP - 1, group, 0)

        i0 = (_NGRP - 1) * _NBUF
        for b in range(_NBUF):
            gather_chunk(i0 + b, b).wait()
            store_chunk(i0 + b, b).start()
        for b in range(_NBUF):
            store_chunk(i0 + b, b).wait()
        for t in range(_NTAIL):
            i = _NGRP * _NBUF + t
            gather_chunk(i, t).start()
        for t in range(_NTAIL):
            i = _NGRP * _NBUF + t
            gather_chunk(i, t).wait()
            store_chunk(i, t).start()
        for t in range(_NTAIL):
            i = _NGRP * _NBUF + t
            store_chunk(i, t).wait()

    return gather_kernel


_gather = _make_gather()


@jax.jit
def _lookup(table, idx2):
    return _gather(table, idx2)


def kernel(input_ids, embed_weight):
    idx_flat = input_ids.reshape(-1).astype(jnp.int32)
    idx2 = idx_flat.reshape(_NW * _NCHUNK, _C)
    out = _lookup(embed_weight, idx2)
    return out.reshape(input_ids.shape + (embed_weight.shape[-1],))
